# baseline (device time: 99963 ns/iter reference)
import functools

import jax
import jax.numpy as jnp
from jax import lax
from jax.experimental import pallas as pl
from jax.experimental.pallas import tpu as pltpu

N_DEV = 4
E_TOTAL = 16
E_LOCAL = E_TOTAL // N_DEV
N_TOK = 2048
D_MODEL = 512
D_HID = 1024
CHUNK = N_TOK // N_DEV


def kernel(x, router_W, route_idx, expert_W, shared_W):
    def body(x_ref, rw_ref, idx_ref, ew_ref, sw_ref, out_ref,
             partial_ref, send_buf, comm_ref, send_sems, recv_sems):
        my = lax.axis_index("i")
        left = lax.rem(my + N_DEV - 1, N_DEV)
        right = lax.rem(my + 1, N_DEV)

        barrier_sem = pltpu.get_barrier_semaphore()
        for nbr in (left, right):
            pl.semaphore_signal(
                barrier_sem, inc=1,
                device_id=(nbr,), device_id_type=pl.DeviceIdType.MESH,
            )
        pl.semaphore_wait(barrier_sem, 2)

        x = x_ref[:, :]
        scores = jnp.dot(x, rw_ref[:, :], preferred_element_type=jnp.float32)
        s_max = jnp.max(scores, axis=-1, keepdims=True)
        p = jnp.exp(scores - s_max)
        probs = p / jnp.sum(p, axis=-1, keepdims=True)
        lane = lax.broadcasted_iota(jnp.int32, (N_TOK, E_TOTAL), 1)
        routed = jnp.where(lane == idx_ref[:, :], probs, 0.0)

        for j in range(E_LOCAL):
            e_id = my * E_LOCAL + j
            w = jnp.sum(
                jnp.where(lane == e_id, routed, 0.0), axis=1, keepdims=True
            )
            contrib = jnp.dot(
                x * w, ew_ref[j], preferred_element_type=jnp.float32
            )
            if j == 0:
                partial_ref[:, :] = contrib
            else:
                partial_ref[:, :] += contrib

        for s in range(N_DEV - 1):
            c = lax.rem(my + 2 * N_DEV - 1 - s, N_DEV)
            own = partial_ref[pl.ds(c * CHUNK, CHUNK), :]
            if s == 0:
                send_buf[:, :] = own
            else:
                send_buf[:, :] = comm_ref[s - 1] + own
            rdma = pltpu.make_async_remote_copy(
                src_ref=send_buf,
                dst_ref=comm_ref.at[s],
                send_sem=send_sems.at[s],
                recv_sem=recv_sems.at[s],
                device_id=(right,),
                device_id_type=pl.DeviceIdType.MESH,
            )
            rdma.start()
            rdma.wait()

        expert_out = (
            comm_ref[N_DEV - 2] + partial_ref[pl.ds(my * CHUNK, CHUNK), :]
        )
        shared = jnp.dot(
            x_ref[pl.ds(my * CHUNK, CHUNK), :],
            sw_ref[:, :],
            preferred_element_type=jnp.float32,
        )
        out_ref[:, :] = shared + expert_out

        @functools.partial(
            pl.run_scoped, second_barrier=pltpu.SemaphoreType.REGULAR
        )
        def _(second_barrier):
            for nbr in (left, right):
                pl.semaphore_signal(
                    second_barrier, inc=1,
                    device_id=(nbr,), device_id_type=pl.DeviceIdType.MESH,
                )
            pl.semaphore_wait(second_barrier, 2)

    return pl.pallas_call(
        body,
        out_shape=jax.ShapeDtypeStruct((CHUNK, D_HID), jnp.float32),
        in_specs=[
            pl.BlockSpec(memory_space=pltpu.VMEM),
            pl.BlockSpec(memory_space=pltpu.VMEM),
            pl.BlockSpec(memory_space=pltpu.VMEM),
            pl.BlockSpec(memory_space=pltpu.VMEM),
            pl.BlockSpec(memory_space=pltpu.VMEM),
        ],
        out_specs=pl.BlockSpec(memory_space=pltpu.VMEM),
        scratch_shapes=[
            pltpu.VMEM((N_TOK, D_HID), jnp.float32),
            pltpu.VMEM((CHUNK, D_HID), jnp.float32),
            pltpu.VMEM((N_DEV - 1, CHUNK, D_HID), jnp.float32),
            pltpu.SemaphoreType.DMA((N_DEV - 1,)),
            pltpu.SemaphoreType.DMA((N_DEV - 1,)),
        ],
        compiler_params=pltpu.CompilerParams(collective_id=0),
    )(x, router_W, route_idx, expert_W, shared_W)


# device time: 50918 ns/iter; 1.9632x vs baseline; 1.9632x over previous
import functools

import jax
import jax.numpy as jnp
from jax import lax
from jax.experimental import pallas as pl
from jax.experimental.pallas import tpu as pltpu

N_DEV = 4
E_TOTAL = 16
E_LOCAL = E_TOTAL // N_DEV
N_TOK = 2048
D_MODEL = 512
D_HID = 1024
CHUNK = N_TOK // N_DEV


def kernel(x, router_W, route_idx, expert_W, shared_W):
    def body(x_ref, rw_ref, idx_ref, ew_ref, sw_ref, out_ref,
             routed_ref, ewb_ref, send_bufs, recv_bufs, send_sems,
             recv_sems):
        my = lax.axis_index("i")

        barrier_sem = pltpu.get_barrier_semaphore()
        for d in range(1, N_DEV):
            pl.semaphore_signal(
                barrier_sem, inc=1,
                device_id=(lax.rem(my + d, N_DEV),),
                device_id_type=pl.DeviceIdType.MESH,
            )
        pl.semaphore_wait(barrier_sem, N_DEV - 1)

        scores = jnp.dot(
            x_ref[:, :], rw_ref[:, :], preferred_element_type=jnp.float32
        )
        s_max = jnp.max(scores, axis=-1, keepdims=True)
        p = jnp.exp(scores - s_max)
        probs = p / jnp.sum(p, axis=-1, keepdims=True)
        lane = lax.broadcasted_iota(jnp.int32, (N_TOK, E_TOTAL), 1)
        routed_ref[:, :] = jnp.where(lane == idx_ref[:, :], probs, 0.0)

        ewb_ref[:, :, :] = ew_ref[:, :, :].astype(jnp.bfloat16)

        def chunk_contrib(c):
            rows = pl.ds(c * CHUNK, CHUNK)
            xc = x_ref[rows, :]
            rc = routed_ref[pl.ds(c * CHUNK, CHUNK), :]
            lc = lane[:CHUNK, :]
            acc = None
            for j in range(E_LOCAL):
                e_id = my * E_LOCAL + j
                w = jnp.sum(
                    jnp.where(lc == e_id, rc, 0.0), axis=1, keepdims=True
                )
                xw = (xc * w).astype(jnp.bfloat16)
                term = jnp.dot(
                    xw, ewb_ref[j], preferred_element_type=jnp.float32
                )
                acc = term if acc is None else acc + term
            return acc

        rdmas = []
        for d in range(1, N_DEV):
            tgt = lax.rem(my + d, N_DEV)
            send_bufs[d - 1, :, :] = chunk_contrib(tgt).astype(jnp.bfloat16)
            rdma = pltpu.make_async_remote_copy(
                src_ref=send_bufs.at[d - 1],
                dst_ref=recv_bufs.at[d - 1],
                send_sem=send_sems.at[d - 1],
                recv_sem=recv_sems.at[d - 1],
                device_id=(tgt,),
                device_id_type=pl.DeviceIdType.MESH,
            )
            rdma.start()
            rdmas.append(rdma)

        acc = chunk_contrib(my)
        acc += jnp.dot(
            x_ref[pl.ds(my * CHUNK, CHUNK), :],
            sw_ref[:, :],
            preferred_element_type=jnp.float32,
        )
        for d in range(1, N_DEV):
            rdmas[d - 1].wait_recv()
            acc += recv_bufs[d - 1].astype(jnp.float32)
        out_ref[:, :] = acc
        for d in range(1, N_DEV):
            rdmas[d - 1].wait_send()

        @functools.partial(
            pl.run_scoped, second_barrier=pltpu.SemaphoreType.REGULAR
        )
        def _(second_barrier):
            for d in range(1, N_DEV):
                pl.semaphore_signal(
                    second_barrier, inc=1,
                    device_id=(lax.rem(my + d, N_DEV),),
                    device_id_type=pl.DeviceIdType.MESH,
                )
            pl.semaphore_wait(second_barrier, N_DEV - 1)

    return pl.pallas_call(
        body,
        out_shape=jax.ShapeDtypeStruct((CHUNK, D_HID), jnp.float32),
        in_specs=[
            pl.BlockSpec(memory_space=pltpu.VMEM),
            pl.BlockSpec(memory_space=pltpu.VMEM),
            pl.BlockSpec(memory_space=pltpu.VMEM),
            pl.BlockSpec(memory_space=pltpu.VMEM),
            pl.BlockSpec(memory_space=pltpu.VMEM),
        ],
        out_specs=pl.BlockSpec(memory_space=pltpu.VMEM),
        scratch_shapes=[
            pltpu.VMEM((N_TOK, E_TOTAL), jnp.float32),
            pltpu.VMEM((E_LOCAL, D_MODEL, D_HID), jnp.bfloat16),
            pltpu.VMEM((N_DEV - 1, CHUNK, D_HID), jnp.bfloat16),
            pltpu.VMEM((N_DEV - 1, CHUNK, D_HID), jnp.bfloat16),
            pltpu.SemaphoreType.DMA((N_DEV - 1,)),
            pltpu.SemaphoreType.DMA((N_DEV - 1,)),
        ],
        compiler_params=pltpu.CompilerParams(collective_id=0),
    )(x, router_W, route_idx, expert_W, shared_W)
